# baseline (speedup 1.0000x reference)
import jax
import jax.numpy as jnp
from jax import lax
from jax.experimental import pallas as pl
from jax.experimental.pallas import tpu as pltpu
from jax.experimental.pallas import tpu_sc as plsc

_B = 16384
_NC = 2
_NS = 16
_NW = _NC * _NS
_BPW = _B // _NW
_G = _BPW // 16


def _pose_body(ind_hbm, rots_hbm, trans_hbm, rot_out, tran_out,
               idx_v, eidx_v, gat_v, r_v, t_v, sem):
    wid = lax.axis_index("s") * _NC + lax.axis_index("c")
    base = wid * _BPW

    pltpu.sync_copy(ind_hbm.at[pl.ds(base, _BPW)], idx_v)

    def build(g, carry):
        v = idx_v[pl.ds(g * 16, 16)]
        q4 = v * 4
        t2 = v * 2
        eidx_v[pl.ds(g * 16, 16)] = q4
        eidx_v[pl.ds(_BPW + g * 16, 16)] = q4 + 1
        eidx_v[pl.ds(2 * _BPW + g * 16, 16)] = q4 + 2
        eidx_v[pl.ds(3 * _BPW + g * 16, 16)] = q4 + 3
        eidx_v[pl.ds(4 * _BPW + g * 16, 16)] = t2
        eidx_v[pl.ds(5 * _BPW + g * 16, 16)] = t2 + 1
        return carry

    lax.fori_loop(0, _G, build, 0)

    copies = []
    for c in range(4):
        sl = pl.ds(c * _BPW, _BPW)
        copies.append(pltpu.async_copy(rots_hbm.at[eidx_v.at[sl]],
                                       gat_v.at[sl], sem))
    for c in range(2):
        sl = pl.ds((4 + c) * _BPW, _BPW)
        copies.append(pltpu.async_copy(trans_hbm.at[eidx_v.at[sl]],
                                       gat_v.at[sl], sem))
    for cp in copies:
        cp.wait()

    lane = lax.iota(jnp.int32, 16)
    zero = jnp.zeros((16,), jnp.int32)
    one = jnp.float32(1.0)

    def group(g, carry):
        rows = g * 16 + lane
        r = gat_v[pl.ds(g * 16, 16)]
        i = gat_v[pl.ds(_BPW + g * 16, 16)]
        j = gat_v[pl.ds(2 * _BPW + g * 16, 16)]
        k = gat_v[pl.ds(3 * _BPW + g * 16, 16)]

        rr, ii, jj, kk = r * r, i * i, j * j, k * k
        s = 2.0 / (rr + ii + jj + kk)
        ij, ik, jk = i * j, i * k, j * k
        ri, rj, rk = r * i, r * j, r * k

        plsc.store_scatter(r_v, [rows, zero], one - s * (jj + kk))
        plsc.store_scatter(r_v, [rows, zero + 1], s * (ij - rk))
        plsc.store_scatter(r_v, [rows, zero + 2], s * (ik + rj))
        plsc.store_scatter(r_v, [rows, zero + 3], s * (ij + rk))
        plsc.store_scatter(r_v, [rows, zero + 4], one - s * (ii + kk))
        plsc.store_scatter(r_v, [rows, zero + 5], s * (jk - ri))
        plsc.store_scatter(r_v, [rows, zero + 6], s * (ik - rj))
        plsc.store_scatter(r_v, [rows, zero + 7], s * (jk + ri))
        plsc.store_scatter(r_v, [rows, zero + 8], one - s * (ii + jj))

        t0 = gat_v[pl.ds(4 * _BPW + g * 16, 16)]
        t1 = gat_v[pl.ds(5 * _BPW + g * 16, 16)]
        plsc.store_scatter(t_v, [rows, zero], t0)
        plsc.store_scatter(t_v, [rows, zero + 1], t1)
        return carry

    lax.fori_loop(0, _G, group, 0)

    pltpu.sync_copy(r_v, rot_out.at[pl.ds(base, _BPW)])
    pltpu.sync_copy(t_v, tran_out.at[pl.ds(base, _BPW)])


_pose_call = pl.kernel(
    _pose_body,
    out_type=[
        jax.ShapeDtypeStruct((_B, 9), jnp.float32),
        jax.ShapeDtypeStruct((_B, 2), jnp.float32),
    ],
    mesh=plsc.VectorSubcoreMesh(core_axis_name="c", subcore_axis_name="s",
                                num_cores=_NC, num_subcores=_NS),
    compiler_params=pltpu.CompilerParams(
        needs_layout_passes=False, use_tc_tiling_on_sc=False),
    scratch_types=[
        pltpu.VMEM((_BPW,), jnp.int32),
        pltpu.VMEM((6 * _BPW,), jnp.int32),
        pltpu.VMEM((6 * _BPW,), jnp.float32),
        pltpu.VMEM((_BPW, 9), jnp.float32),
        pltpu.VMEM((_BPW, 2), jnp.float32),
        pltpu.SemaphoreType.DMA,
    ],
)


@jax.jit
def kernel(ind, rots_emb_weight, trans_emb_weight):
    rot9, tran = _pose_call(ind.astype(jnp.int32),
                            rots_emb_weight.reshape(-1),
                            trans_emb_weight.reshape(-1))
    return rot9.reshape(_B, 3, 3), tran


# speedup vs baseline: 16.0983x; 16.0983x over previous
import jax
import jax.numpy as jnp
from jax import lax
from jax.experimental import pallas as pl
from jax.experimental.pallas import tpu as pltpu
from jax.experimental.pallas import tpu_sc as plsc

_B = 16384
_NC = 2
_NS = 16
_NW = _NC * _NS
_BPW = _B // _NW
_G = _BPW // 16


def _pose_body(ind_hbm, qr_hbm, qi_hbm, qj_hbm, qk_hbm, t0_hbm, t1_hbm,
               rot_out, tran_out, idx_v, gat_v, r_v, t_v, sem):
    wid = lax.axis_index("s") * _NC + lax.axis_index("c")
    base = wid * _BPW

    pltpu.sync_copy(ind_hbm.at[pl.ds(base, _BPW)], idx_v)

    tables = (qr_hbm, qi_hbm, qj_hbm, qk_hbm, t0_hbm, t1_hbm)
    copies = [
        pltpu.async_copy(tab.at[idx_v], gat_v.at[pl.ds(c * _BPW, _BPW)], sem)
        for c, tab in enumerate(tables)
    ]
    for cp in copies:
        cp.wait()

    lane = lax.iota(jnp.int32, 16)
    zero = jnp.zeros((16,), jnp.int32)
    one = jnp.float32(1.0)

    def group(g, carry):
        rows = g * 16 + lane
        r = gat_v[pl.ds(g * 16, 16)]
        i = gat_v[pl.ds(_BPW + g * 16, 16)]
        j = gat_v[pl.ds(2 * _BPW + g * 16, 16)]
        k = gat_v[pl.ds(3 * _BPW + g * 16, 16)]

        rr, ii, jj, kk = r * r, i * i, j * j, k * k
        s = 2.0 / (rr + ii + jj + kk)
        ij, ik, jk = i * j, i * k, j * k
        ri, rj, rk = r * i, r * j, r * k

        plsc.store_scatter(r_v, [rows, zero], one - s * (jj + kk))
        plsc.store_scatter(r_v, [rows, zero + 1], s * (ij - rk))
        plsc.store_scatter(r_v, [rows, zero + 2], s * (ik + rj))
        plsc.store_scatter(r_v, [rows, zero + 3], s * (ij + rk))
        plsc.store_scatter(r_v, [rows, zero + 4], one - s * (ii + kk))
        plsc.store_scatter(r_v, [rows, zero + 5], s * (jk - ri))
        plsc.store_scatter(r_v, [rows, zero + 6], s * (ik - rj))
        plsc.store_scatter(r_v, [rows, zero + 7], s * (jk + ri))
        plsc.store_scatter(r_v, [rows, zero + 8], one - s * (ii + jj))

        t0 = gat_v[pl.ds(4 * _BPW + g * 16, 16)]
        t1 = gat_v[pl.ds(5 * _BPW + g * 16, 16)]
        plsc.store_scatter(t_v, [rows, zero], t0)
        plsc.store_scatter(t_v, [rows, zero + 1], t1)
        return carry

    lax.fori_loop(0, _G, group, 0)

    pltpu.sync_copy(r_v, rot_out.at[pl.ds(base, _BPW)])
    pltpu.sync_copy(t_v, tran_out.at[pl.ds(base, _BPW)])


_pose_call = pl.kernel(
    _pose_body,
    out_type=[
        jax.ShapeDtypeStruct((_B, 9), jnp.float32),
        jax.ShapeDtypeStruct((_B, 2), jnp.float32),
    ],
    mesh=plsc.VectorSubcoreMesh(core_axis_name="c", subcore_axis_name="s",
                                num_cores=_NC, num_subcores=_NS),
    compiler_params=pltpu.CompilerParams(
        needs_layout_passes=False, use_tc_tiling_on_sc=False),
    scratch_types=[
        pltpu.VMEM((_BPW,), jnp.int32),
        pltpu.VMEM((6 * _BPW,), jnp.float32),
        pltpu.VMEM((_BPW, 9), jnp.float32),
        pltpu.VMEM((_BPW, 2), jnp.float32),
        pltpu.SemaphoreType.DMA,
    ],
)


@jax.jit
def kernel(ind, rots_emb_weight, trans_emb_weight):
    rot9, tran = _pose_call(
        ind.astype(jnp.int32),
        rots_emb_weight[:, 0], rots_emb_weight[:, 1],
        rots_emb_weight[:, 2], rots_emb_weight[:, 3],
        trans_emb_weight[:, 0], trans_emb_weight[:, 1],
    )
    return rot9.reshape(_B, 3, 3), tran


# speedup vs baseline: 18.6067x; 1.1558x over previous
import jax
import jax.numpy as jnp
from jax import lax
from jax.experimental import pallas as pl
from jax.experimental.pallas import tpu as pltpu
from jax.experimental.pallas import tpu_sc as plsc

_B = 16384
_NC = 2
_NS = 16
_NW = _NC * _NS
_BPW = _B // _NW
_G = _BPW // 16


def _pose_body(ind_hbm, qr_hbm, qi_hbm, qj_hbm, qk_hbm, t0_hbm, t1_hbm,
               *rest):
    outs = rest[:11]
    idx_v, gat_v, out_v, sem, osem = rest[11:]
    wid = lax.axis_index("s") * _NC + lax.axis_index("c")
    base = wid * _BPW

    pltpu.sync_copy(ind_hbm.at[pl.ds(base, _BPW)], idx_v)

    tables = (qr_hbm, qi_hbm, qj_hbm, qk_hbm, t0_hbm, t1_hbm)
    copies = [
        pltpu.async_copy(tab.at[idx_v], gat_v.at[pl.ds(c * _BPW, _BPW)], sem)
        for c, tab in enumerate(tables)
    ]
    for cp in copies:
        cp.wait()

    one = jnp.float32(1.0)

    def group(g, carry):
        sl = pl.ds(g * 16, 16)
        r = gat_v[sl]
        i = gat_v[pl.ds(_BPW + g * 16, 16)]
        j = gat_v[pl.ds(2 * _BPW + g * 16, 16)]
        k = gat_v[pl.ds(3 * _BPW + g * 16, 16)]

        rr, ii, jj, kk = r * r, i * i, j * j, k * k
        s = 2.0 / (rr + ii + jj + kk)
        ij, ik, jk = i * j, i * k, j * k
        ri, rj, rk = r * i, r * j, r * k

        vals = (one - s * (jj + kk), s * (ij - rk), s * (ik + rj),
                s * (ij + rk), one - s * (ii + kk), s * (jk - ri),
                s * (ik - rj), s * (jk + ri), one - s * (ii + jj),
                gat_v[pl.ds(4 * _BPW + g * 16, 16)],
                gat_v[pl.ds(5 * _BPW + g * 16, 16)])
        for c, v in enumerate(vals):
            out_v[pl.ds(c * _BPW + g * 16, 16)] = v
        return carry

    lax.fori_loop(0, _G, group, 0)

    ocopies = [
        pltpu.async_copy(out_v.at[pl.ds(c * _BPW, _BPW)],
                         o.at[pl.ds(base, _BPW)], osem)
        for c, o in enumerate(outs)
    ]
    for cp in ocopies:
        cp.wait()


_pose_call = pl.kernel(
    _pose_body,
    out_type=[jax.ShapeDtypeStruct((_B,), jnp.float32)] * 11,
    mesh=plsc.VectorSubcoreMesh(core_axis_name="c", subcore_axis_name="s",
                                num_cores=_NC, num_subcores=_NS),
    compiler_params=pltpu.CompilerParams(
        needs_layout_passes=False, use_tc_tiling_on_sc=False),
    scratch_types=[
        pltpu.VMEM((_BPW,), jnp.int32),
        pltpu.VMEM((6 * _BPW,), jnp.float32),
        pltpu.VMEM((11 * _BPW,), jnp.float32),
        pltpu.SemaphoreType.DMA,
        pltpu.SemaphoreType.DMA,
    ],
)


@jax.jit
def kernel(ind, rots_emb_weight, trans_emb_weight):
    outs = _pose_call(
        ind.astype(jnp.int32),
        rots_emb_weight[:, 0], rots_emb_weight[:, 1],
        rots_emb_weight[:, 2], rots_emb_weight[:, 3],
        trans_emb_weight[:, 0], trans_emb_weight[:, 1],
    )
    rot = jnp.stack(outs[:9], axis=1).reshape(_B, 3, 3)
    tran = jnp.stack(outs[9:], axis=1)
    return rot, tran


# speedup vs baseline: 19.4060x; 1.0430x over previous
import jax
import jax.numpy as jnp
from jax import lax
from jax.experimental import pallas as pl
from jax.experimental.pallas import tpu as pltpu
from jax.experimental.pallas import tpu_sc as plsc

_B = 16384
_NC = 2
_NS = 16
_NW = _NC * _NS
_BPW = _B // _NW
_G = _BPW // 16

_MESH = plsc.VectorSubcoreMesh(core_axis_name="c", subcore_axis_name="s",
                               num_cores=_NC, num_subcores=_NS)
_CPARAMS = pltpu.CompilerParams(
    needs_layout_passes=False, use_tc_tiling_on_sc=False)


def _tran_body(ind_hbm, t0_hbm, t1_hbm, o0_hbm, o1_hbm, idx_v, gat_v, sem):
    wid = lax.axis_index("s") * _NC + lax.axis_index("c")
    base = wid * _BPW
    pltpu.sync_copy(ind_hbm.at[pl.ds(base, _BPW)], idx_v)
    c0 = pltpu.async_copy(t0_hbm.at[idx_v], gat_v.at[pl.ds(0, _BPW)], sem)
    c1 = pltpu.async_copy(t1_hbm.at[idx_v], gat_v.at[pl.ds(_BPW, _BPW)], sem)
    c0.wait()
    c1.wait()
    o0 = pltpu.async_copy(gat_v.at[pl.ds(0, _BPW)],
                          o0_hbm.at[pl.ds(base, _BPW)], sem)
    o1 = pltpu.async_copy(gat_v.at[pl.ds(_BPW, _BPW)],
                          o1_hbm.at[pl.ds(base, _BPW)], sem)
    o0.wait()
    o1.wait()


_tran_call = pl.kernel(
    _tran_body,
    out_type=[jax.ShapeDtypeStruct((_B,), jnp.float32)] * 2,
    mesh=_MESH,
    compiler_params=_CPARAMS,
    scratch_types=[
        pltpu.VMEM((_BPW,), jnp.int32),
        pltpu.VMEM((2 * _BPW,), jnp.float32),
        pltpu.SemaphoreType.DMA,
    ],
)


def _quat_body(ind_hbm, qr_hbm, qi_hbm, qj_hbm, qk_hbm, *rest):
    outs = rest[:9]
    idx_v, gat_v, out_v, sem, osem = rest[9:]
    wid = lax.axis_index("s") * _NC + lax.axis_index("c")
    base = wid * _BPW

    pltpu.sync_copy(ind_hbm.at[pl.ds(base, _BPW)], idx_v)
    tables = (qr_hbm, qi_hbm, qj_hbm, qk_hbm)
    copies = [
        pltpu.async_copy(tab.at[idx_v], gat_v.at[pl.ds(c * _BPW, _BPW)], sem)
        for c, tab in enumerate(tables)
    ]
    for cp in copies:
        cp.wait()

    one = jnp.float32(1.0)

    def group(g, carry):
        r = gat_v[pl.ds(g * 16, 16)]
        i = gat_v[pl.ds(_BPW + g * 16, 16)]
        j = gat_v[pl.ds(2 * _BPW + g * 16, 16)]
        k = gat_v[pl.ds(3 * _BPW + g * 16, 16)]

        rr, ii, jj, kk = r * r, i * i, j * j, k * k
        s = 2.0 / (rr + ii + jj + kk)
        ij, ik, jk = i * j, i * k, j * k
        ri, rj, rk = r * i, r * j, r * k

        vals = (one - s * (jj + kk), s * (ij - rk), s * (ik + rj),
                s * (ij + rk), one - s * (ii + kk), s * (jk - ri),
                s * (ik - rj), s * (jk + ri), one - s * (ii + jj))
        for c, v in enumerate(vals):
            out_v[pl.ds(c * _BPW + g * 16, 16)] = v
        return carry

    lax.fori_loop(0, _G, group, 0)

    ocopies = [
        pltpu.async_copy(out_v.at[pl.ds(c * _BPW, _BPW)],
                         o.at[pl.ds(base, _BPW)], osem)
        for c, o in enumerate(outs)
    ]
    for cp in ocopies:
        cp.wait()


_quat_call = pl.kernel(
    _quat_body,
    out_type=[jax.ShapeDtypeStruct((_B,), jnp.float32)] * 9,
    mesh=_MESH,
    compiler_params=_CPARAMS,
    scratch_types=[
        pltpu.VMEM((_BPW,), jnp.int32),
        pltpu.VMEM((4 * _BPW,), jnp.float32),
        pltpu.VMEM((9 * _BPW,), jnp.float32),
        pltpu.SemaphoreType.DMA,
        pltpu.SemaphoreType.DMA,
    ],
)


@jax.jit
def kernel(ind, rots_emb_weight, trans_emb_weight):
    ind32 = ind.astype(jnp.int32)
    t0, t1 = _tran_call(ind32, trans_emb_weight[:, 0], trans_emb_weight[:, 1])
    mats = _quat_call(
        ind32,
        rots_emb_weight[:, 0], rots_emb_weight[:, 1],
        rots_emb_weight[:, 2], rots_emb_weight[:, 3],
    )
    rot = jnp.stack(mats, axis=1).reshape(_B, 3, 3)
    tran = jnp.stack([t0, t1], axis=1)
    return rot, tran
